# baseline (speedup 1.0000x reference)
import functools
import math

import jax
import jax.numpy as jnp
from jax import lax
from jax.experimental import pallas as pl
from jax.experimental.pallas import tpu as pltpu
from jax.experimental.pallas import tpu_sc as plsc

D = 64
SCALE = math.sqrt(D)

NC = 2
NS = 16
NW = NC * NS
L = 16

NJ = 200
NI = 4096
BI = 128
FH = 8
FL = 8

_mesh = plsc.VectorSubcoreMesh(core_axis_name="c", subcore_axis_name="s")


@functools.partial(
    pl.kernel,
    mesh=_mesh,
    out_type=jax.ShapeDtypeStruct((NJ, FH, NW, FL, BI), jnp.float32),
    scratch_types=[
        pltpu.VMEM((NJ, BI), jnp.int32),
        pltpu.VMEM((BI, D), jnp.float32),
        pltpu.VMEM((BI, D), jnp.float32),
        pltpu.VMEM((FH, FL, BI), jnp.float32),
        pltpu.VMEM((FH, FL, BI), jnp.float32),
        pltpu.SemaphoreType.DMA,
        pltpu.SemaphoreType.DMA,
        pltpu.SemaphoreType.DMA,
        pltpu.SemaphoreType.DMA,
    ],
    compiler_params=pltpu.CompilerParams(
        use_tc_tiling_on_sc=False, needs_layout_passes=False),
)
def _embed(tok3d_hbm, table_hbm, out_hbm,
           idx_v, gb0, gb1, wb0, wb1, gs0, gs1, ws0, ws1):
    gbuf = (gb0, gb1)
    wbuf = (wb0, wb1)
    gsem = (gs0, gs1)
    wsem = (ws0, ws1)

    w = lax.axis_index("s") * NC + lax.axis_index("c")

    pltpu.sync_copy(tok3d_hbm.at[:, w, :], idx_v)

    iota = lax.iota(jnp.int32, L)

    def gdesc(j, b):
        return pltpu.make_async_copy(
            table_hbm.at[idx_v.at[j]], gbuf[b], gsem[b])

    def wdesc(j, b):
        return pltpu.make_async_copy(
            wbuf[b], out_hbm.at[j, :, w], wsem[b])

    def transpose_scale(b):
        src = gbuf[b]
        dst = wbuf[b]

        @plsc.parallel_loop(0, FH * FL, unroll=4)
        def _(q):
            fh = q // FL
            fl = q % FL
            col = jnp.full((L,), q, jnp.int32)
            for k in range(BI // L):
                rows = iota + (k * L)
                vals = plsc.load_gather(src, [rows, col]) * SCALE
                dst[fh, fl, pl.ds(k * L, L)] = vals

    gdesc(0, 0).start()
    gdesc(1, 1).start()

    for b in range(2):
        gdesc(b, b).wait()
        transpose_scale(b)
        gdesc(b + 2, b).start()
        wdesc(b, b).start()

    def pair_body(k, carry):
        for b in range(2):
            j = 2 * k + b
            gdesc(j, b).wait()
            wdesc(j - 2, b).wait()
            transpose_scale(b)
            gdesc(j + 2, b).start()
            wdesc(j, b).start()
        return carry

    lax.fori_loop(1, NJ // 2 - 1, pair_body, 0)

    for b in range(2):
        j = NJ - 2 + b
        gdesc(j, b).wait()
        wdesc(j - 2, b).wait()
        transpose_scale(b)
        wdesc(j, b).start()

    for b in range(2):
        wdesc(NJ - 2 + b, b).wait()


def kernel(tokens, table):
    tok3d = tokens.T.reshape(NJ, NW, BI)
    out5 = _embed(tok3d, table)
    return out5.transpose(2, 4, 0, 1, 3).reshape(NI, NJ, D)


# speedup vs baseline: 1.6158x; 1.6158x over previous
import functools
import math

import jax
import jax.numpy as jnp
from jax import lax
from jax.experimental import pallas as pl
from jax.experimental.pallas import tpu as pltpu
from jax.experimental.pallas import tpu_sc as plsc

D = 64
SCALE = math.sqrt(D)

NC = 2
NS = 16
NW = NC * NS
L = 16

NJ = 200
NI = 4096
BI = 128
FH = 8
FL = 8
PITCH = BI + 1

_mesh = plsc.VectorSubcoreMesh(core_axis_name="c", subcore_axis_name="s")


@functools.partial(
    pl.kernel,
    mesh=_mesh,
    out_type=jax.ShapeDtypeStruct((NJ, FH, NW, FL, BI), jnp.float32),
    scratch_types=[
        pltpu.VMEM((NJ, BI), jnp.int32),
        pltpu.VMEM((BI, D), jnp.float32),
        pltpu.VMEM((BI, D), jnp.float32),
        pltpu.VMEM((FH, FL, PITCH), jnp.float32),
        pltpu.VMEM((FH, FL, PITCH), jnp.float32),
        pltpu.SemaphoreType.DMA,
        pltpu.SemaphoreType.DMA,
        pltpu.SemaphoreType.DMA,
        pltpu.SemaphoreType.DMA,
    ],
    compiler_params=pltpu.CompilerParams(
        use_tc_tiling_on_sc=False, needs_layout_passes=False),
)
def _embed(tok3d_hbm, table_hbm, out_hbm,
           idx_v, gb0, gb1, wb0, wb1, gs0, gs1, ws0, ws1):
    gbuf = (gb0, gb1)
    wbuf = (wb0, wb1)
    gsem = (gs0, gs1)
    wsem = (ws0, ws1)

    w = lax.axis_index("s") * NC + lax.axis_index("c")

    pltpu.sync_copy(tok3d_hbm.at[:, w, :], idx_v)

    iota = lax.iota(jnp.int32, L)
    fhv = []
    flv = []
    for k in range(D // L):
        fvec = iota + (k * L)
        fhv.append(lax.shift_right_logical(fvec, 3))
        flv.append(lax.bitwise_and(fvec, 7))

    def gdesc(j, b):
        return pltpu.make_async_copy(
            table_hbm.at[idx_v.at[j]], gbuf[b], gsem[b])

    def wdesc(j, b):
        return pltpu.make_async_copy(
            wbuf[b].at[:, :, pl.ds(0, BI)], out_hbm.at[j, :, w], wsem[b])

    def transpose_scale(b):
        src = gbuf[b]
        dst = wbuf[b]

        @plsc.parallel_loop(0, BI, unroll=4)
        def _(i):
            coli = jnp.full((L,), i, jnp.int32)
            for k in range(D // L):
                vals = src[i, pl.ds(k * L, L)] * SCALE
                plsc.store_scatter(dst, [fhv[k], flv[k], coli], vals)

    gdesc(0, 0).start()
    gdesc(1, 1).start()

    for b in range(2):
        gdesc(b, b).wait()
        transpose_scale(b)
        gdesc(b + 2, b).start()
        wdesc(b, b).start()

    def pair_body(k, carry):
        for b in range(2):
            j = 2 * k + b
            gdesc(j, b).wait()
            wdesc(j - 2, b).wait()
            transpose_scale(b)
            gdesc(j + 2, b).start()
            wdesc(j, b).start()
        return carry

    lax.fori_loop(1, NJ // 2 - 1, pair_body, 0)

    for b in range(2):
        j = NJ - 2 + b
        gdesc(j, b).wait()
        wdesc(j - 2, b).wait()
        transpose_scale(b)
        wdesc(j, b).start()

    for b in range(2):
        wdesc(NJ - 2 + b, b).wait()


def kernel(tokens, table):
    tok3d = tokens.T.reshape(NJ, NW, BI)
    out5 = _embed(tok3d, table)
    return out5.transpose(2, 4, 0, 1, 3).reshape(NI, NJ, D)


# speedup vs baseline: 1.6181x; 1.0015x over previous
import functools
import math

import jax
import jax.numpy as jnp
from jax import lax
from jax.experimental import pallas as pl
from jax.experimental.pallas import tpu as pltpu
from jax.experimental.pallas import tpu_sc as plsc
from jax.experimental.layout import Format, Layout, with_layout_constraint

D = 64
SCALE = math.sqrt(D)

NC = 2
NS = 16
NW = NC * NS
L = 16

NJ = 200
NI = 4096
BI = 128
FH = 8
FL = 8
PITCH = BI + 1

_mesh = plsc.VectorSubcoreMesh(core_axis_name="c", subcore_axis_name="s")


@functools.partial(
    pl.kernel,
    mesh=_mesh,
    out_type=jax.ShapeDtypeStruct((NJ, FH, NW, FL, BI), jnp.float32),
    scratch_types=[
        pltpu.VMEM((NJ, BI), jnp.int32),
        pltpu.VMEM((BI, D), jnp.float32),
        pltpu.VMEM((BI, D), jnp.float32),
        pltpu.VMEM((FH, FL, PITCH), jnp.float32),
        pltpu.VMEM((FH, FL, PITCH), jnp.float32),
        pltpu.SemaphoreType.DMA,
        pltpu.SemaphoreType.DMA,
        pltpu.SemaphoreType.DMA,
        pltpu.SemaphoreType.DMA,
    ],
    compiler_params=pltpu.CompilerParams(
        use_tc_tiling_on_sc=False, needs_layout_passes=False),
)
def _embed(tok3d_hbm, table_hbm, out_hbm,
           idx_v, gb0, gb1, wb0, wb1, gs0, gs1, ws0, ws1):
    gbuf = (gb0, gb1)
    wbuf = (wb0, wb1)
    gsem = (gs0, gs1)
    wsem = (ws0, ws1)

    w = lax.axis_index("s") * NC + lax.axis_index("c")

    pltpu.sync_copy(tok3d_hbm.at[:, w, :], idx_v)

    iota = lax.iota(jnp.int32, L)
    fhv = []
    flv = []
    for k in range(D // L):
        fvec = iota + (k * L)
        fhv.append(lax.shift_right_logical(fvec, 3))
        flv.append(lax.bitwise_and(fvec, 7))

    def gdesc(j, b):
        return pltpu.make_async_copy(
            table_hbm.at[idx_v.at[j]], gbuf[b], gsem[b])

    def wdesc(j, b):
        return pltpu.make_async_copy(
            wbuf[b].at[:, :, pl.ds(0, BI)], out_hbm.at[j, :, w], wsem[b])

    def transpose_scale(b):
        src = gbuf[b]
        dst = wbuf[b]

        @plsc.parallel_loop(0, BI, unroll=4)
        def _(i):
            coli = jnp.full((L,), i, jnp.int32)
            for k in range(D // L):
                vals = src[i, pl.ds(k * L, L)] * SCALE
                plsc.store_scatter(dst, [fhv[k], flv[k], coli], vals)

    gdesc(0, 0).start()
    gdesc(1, 1).start()

    for b in range(2):
        gdesc(b, b).wait()
        transpose_scale(b)
        gdesc(b + 2, b).start()
        wdesc(b, b).start()

    def pair_body(k, carry):
        for b in range(2):
            j = 2 * k + b
            gdesc(j, b).wait()
            wdesc(j - 2, b).wait()
            transpose_scale(b)
            gdesc(j + 2, b).start()
            wdesc(j, b).start()
        return carry

    lax.fori_loop(1, NJ // 2 - 1, pair_body, 0)

    for b in range(2):
        j = NJ - 2 + b
        gdesc(j, b).wait()
        wdesc(j - 2, b).wait()
        transpose_scale(b)
        wdesc(j, b).start()

    for b in range(2):
        wdesc(NJ - 2 + b, b).wait()


def kernel(tokens, table):
    tok3d = tokens.T.reshape(NJ, NW, BI)
    table_lin = jax.device_put(
        table,
        Format(Layout(major_to_minor=(0, 1), tiling=((8,),)),
               jax.sharding.SingleDeviceSharding(jax.devices()[0])))
    out5 = _embed(tok3d, table_lin)
    return out5.transpose(2, 4, 0, 1, 3).reshape(NI, NJ, D)
